# baseline (speedup 1.0000x reference)
import jax
import jax.numpy as jnp
from jax.experimental import pallas as pl
from jax.experimental.pallas import tpu as pltpu

_PAR = pltpu.CompilerParams(dimension_semantics=("parallel",))

N = 4096
HID = 64
O = 128
THRESH = 0.6
BLK = 512
SBLK = 256


def _u_kernel(f1_ref, f2_ref, w11_ref, w21_ref, u1_ref, u2_ref):
    u1_ref[...] = jnp.dot(f1_ref[...], w11_ref[...],
                          preferred_element_type=jnp.float32)
    u2_ref[...] = jnp.dot(f2_ref[...], w21_ref[...],
                          preferred_element_type=jnp.float32)


def _u(feat1, feat2, e1w1, e2w1):
    d1 = feat1.shape[1]
    d2 = feat2.shape[1]
    return pl.pallas_call(
        _u_kernel,
        grid=(N // BLK,),
        in_specs=[
            pl.BlockSpec((BLK, d1), lambda i: (i, 0)),
            pl.BlockSpec((BLK, d2), lambda i: (i, 0)),
            pl.BlockSpec((d1, HID), lambda i: (0, 0)),
            pl.BlockSpec((d2, HID), lambda i: (0, 0)),
        ],
        out_specs=[
            pl.BlockSpec((BLK, HID), lambda i: (i, 0)),
            pl.BlockSpec((BLK, HID), lambda i: (i, 0)),
        ],
        out_shape=[
            jax.ShapeDtypeStruct((N, HID), jnp.float32),
            jax.ShapeDtypeStruct((N, HID), jnp.float32),
        ],
        compiler_params=_PAR,
    )(feat1, feat2, e1w1, e2w1)


def _pre_body(adj, u, w2):
    h = jnp.dot(adj, u, preferred_element_type=jnp.float32)
    h = jnp.maximum(h, 0.0)
    norm = jnp.sqrt(jnp.sum(h * h, axis=1, keepdims=True))
    hn = h / jnp.maximum(norm, 1e-12)
    yin = jnp.dot(h, w2, preferred_element_type=jnp.float32)
    return hn, yin


def _pre2_kernel(a1_ref, a2_ref, u1_ref, u2_ref, w12_ref, w22_ref,
                 hn1_ref, yin1_ref, hn2_ref, yin2_ref):
    hn1_ref[...], yin1_ref[...] = _pre_body(a1_ref[...], u1_ref[...],
                                            w12_ref[...])
    hn2_ref[...], yin2_ref[...] = _pre_body(a2_ref[...], u2_ref[...],
                                            w22_ref[...])


def _pre2(adj1, adj2, u1, u2, e1w2, e2w2):
    return pl.pallas_call(
        _pre2_kernel,
        grid=(N // BLK,),
        in_specs=[
            pl.BlockSpec((BLK, N), lambda i: (i, 0)),
            pl.BlockSpec((BLK, N), lambda i: (i, 0)),
            pl.BlockSpec((N, HID), lambda i: (0, 0)),
            pl.BlockSpec((N, HID), lambda i: (0, 0)),
            pl.BlockSpec((HID, O), lambda i: (0, 0)),
            pl.BlockSpec((HID, O), lambda i: (0, 0)),
        ],
        out_specs=[
            pl.BlockSpec((BLK, HID), lambda i: (i, 0)),
            pl.BlockSpec((BLK, O), lambda i: (i, 0)),
            pl.BlockSpec((BLK, HID), lambda i: (i, 0)),
            pl.BlockSpec((BLK, O), lambda i: (i, 0)),
        ],
        out_shape=[
            jax.ShapeDtypeStruct((N, HID), jnp.float32),
            jax.ShapeDtypeStruct((N, O), jnp.float32),
            jax.ShapeDtypeStruct((N, HID), jnp.float32),
            jax.ShapeDtypeStruct((N, O), jnp.float32),
        ],
        compiler_params=_PAR,
    )(adj1, adj2, u1, u2, e1w2, e2w2)


def _simagg_body(hnb, hn, yin):
    s = jax.lax.dot_general(
        hnb, hn,
        dimension_numbers=(((1,), (1,)), ((), ())),
        preferred_element_type=jnp.float32)
    s = jnp.where(s < THRESH, 0.0, s)
    rs = jnp.sum(s, axis=1, keepdims=True)
    agg = jnp.dot(s, yin, preferred_element_type=jnp.float32)
    return agg / jnp.maximum(rs, 1e-12)


def _simagg2_kernel(hnb1_ref, hn1_ref, yin1_ref, hnb2_ref, hn2_ref, yin2_ref,
                    y1_ref, y2_ref):
    y1_ref[...] = _simagg_body(hnb1_ref[...], hn1_ref[...], yin1_ref[...])
    y2_ref[...] = _simagg_body(hnb2_ref[...], hn2_ref[...], yin2_ref[...])


def _simagg2(hn1, yin1, hn2, yin2):
    return pl.pallas_call(
        _simagg2_kernel,
        grid=(N // SBLK,),
        in_specs=[
            pl.BlockSpec((SBLK, HID), lambda i: (i, 0)),
            pl.BlockSpec((N, HID), lambda i: (0, 0)),
            pl.BlockSpec((N, O), lambda i: (0, 0)),
            pl.BlockSpec((SBLK, HID), lambda i: (i, 0)),
            pl.BlockSpec((N, HID), lambda i: (0, 0)),
            pl.BlockSpec((N, O), lambda i: (0, 0)),
        ],
        out_specs=[
            pl.BlockSpec((SBLK, O), lambda i: (i, 0)),
            pl.BlockSpec((SBLK, O), lambda i: (i, 0)),
        ],
        out_shape=[
            jax.ShapeDtypeStruct((N, O), jnp.float32),
            jax.ShapeDtypeStruct((N, O), jnp.float32),
        ],
        compiler_params=_PAR,
    )(hn1, hn1, yin1, hn2, hn2, yin2)


def _xz_kernel(y1_ref, y2_ref, d1_ref, d2_ref, x1_ref, x2_ref, z_ref):
    y1 = y1_ref[...]
    y2 = y2_ref[...]
    x1_ref[...] = jnp.dot(y1, d1_ref[...], preferred_element_type=jnp.float32)
    x2_ref[...] = jnp.dot(y2, d2_ref[...], preferred_element_type=jnp.float32)
    z_ref[...] = (y1 + y2) * 0.5


def _xz(y1, y2, d1w, d2w):
    d1o = d1w.shape[1]
    d2o = d2w.shape[1]
    return pl.pallas_call(
        _xz_kernel,
        grid=(N // BLK,),
        in_specs=[
            pl.BlockSpec((BLK, O), lambda i: (i, 0)),
            pl.BlockSpec((BLK, O), lambda i: (i, 0)),
            pl.BlockSpec((O, d1o), lambda i: (0, 0)),
            pl.BlockSpec((O, d2o), lambda i: (0, 0)),
        ],
        out_specs=[
            pl.BlockSpec((BLK, d1o), lambda i: (i, 0)),
            pl.BlockSpec((BLK, d2o), lambda i: (i, 0)),
            pl.BlockSpec((BLK, O), lambda i: (i, 0)),
        ],
        out_shape=[
            jax.ShapeDtypeStruct((N, d1o), jnp.float32),
            jax.ShapeDtypeStruct((N, d2o), jnp.float32),
            jax.ShapeDtypeStruct((N, O), jnp.float32),
        ],
        compiler_params=_PAR,
    )(y1, y2, d1w, d2w)


def _dec2_kernel(a1_ref, a2_ref, x1_ref, x2_ref, r1_ref, r2_ref):
    r1_ref[...] = jnp.dot(a1_ref[...], x1_ref[...],
                          preferred_element_type=jnp.float32)
    r2_ref[...] = jnp.dot(a2_ref[...], x2_ref[...],
                          preferred_element_type=jnp.float32)


def _dec2(adj1, adj2, x1, x2):
    d1 = x1.shape[1]
    d2 = x2.shape[1]
    return pl.pallas_call(
        _dec2_kernel,
        grid=(N // BLK,),
        in_specs=[
            pl.BlockSpec((BLK, N), lambda i: (i, 0)),
            pl.BlockSpec((BLK, N), lambda i: (i, 0)),
            pl.BlockSpec((N, d1), lambda i: (0, 0)),
            pl.BlockSpec((N, d2), lambda i: (0, 0)),
        ],
        out_specs=[
            pl.BlockSpec((BLK, d1), lambda i: (i, 0)),
            pl.BlockSpec((BLK, d2), lambda i: (i, 0)),
        ],
        out_shape=[
            jax.ShapeDtypeStruct((N, d1), jnp.float32),
            jax.ShapeDtypeStruct((N, d2), jnp.float32),
        ],
        compiler_params=_PAR,
    )(adj1, adj2, x1, x2)


def kernel(feat1, feat2, adj_spatial1, adj_spatial2,
           e1w1, e1w2, e2w1, e2w2, d1w, d2w):
    u1, u2 = _u(feat1, feat2, e1w1, e2w1)
    hn1, yin1, hn2, yin2 = _pre2(adj_spatial1, adj_spatial2, u1, u2,
                                 e1w2, e2w2)
    y1, y2 = _simagg2(hn1, yin1, hn2, yin2)
    x1, x2, z = _xz(y1, y2, d1w, d2w)
    recon1, recon2 = _dec2(adj_spatial1, adj_spatial2, x1, x2)
    return (y1, y2, z, recon1, recon2)


# speedup vs baseline: 1.0915x; 1.0915x over previous
import jax
import jax.numpy as jnp
from jax.experimental import pallas as pl
from jax.experimental.pallas import tpu as pltpu

N = 4096
NH = N // 2
HID = 64
O = 128
THRESH = 0.6
ABLK = 256
SBLK = 256
CBLK = 512
NA = N // ABLK
NB = N // SBLK
NC = N // CBLK


def _u_kernel(f1_ref, f2_ref, w11_ref, w21_ref, u1_ref, u2_ref):
    u1_ref[...] = jnp.dot(f1_ref[...], w11_ref[...],
                          preferred_element_type=jnp.float32)
    u2_ref[...] = jnp.dot(f2_ref[...], w21_ref[...],
                          preferred_element_type=jnp.float32)


def _u(feat1, feat2, e1w1, e2w1):
    d1 = feat1.shape[1]
    d2 = feat2.shape[1]
    blk = 512
    return pl.pallas_call(
        _u_kernel,
        grid=(N // blk,),
        in_specs=[
            pl.BlockSpec((blk, d1), lambda i: (i, 0)),
            pl.BlockSpec((blk, d2), lambda i: (i, 0)),
            pl.BlockSpec((d1, HID), lambda i: (0, 0)),
            pl.BlockSpec((d2, HID), lambda i: (0, 0)),
        ],
        out_specs=[
            pl.BlockSpec((blk, HID), lambda i: (i, 0)),
            pl.BlockSpec((blk, HID), lambda i: (i, 0)),
        ],
        out_shape=[
            jax.ShapeDtypeStruct((N, HID), jnp.float32),
            jax.ShapeDtypeStruct((N, HID), jnp.float32),
        ],
    )(feat1, feat2, e1w1, e2w1)


def _chain_body(a_ref, u_ref, w2_ref, dw_ref, yprev_ref,
                y_ref, recon_ref, z_ref,
                adjbf_ref, hn_ref, yin_ref, x_ref):
    i = pl.program_id(0)

    @pl.when(i < NA)
    def _phase_a():
        a = a_ref[...]
        h = jnp.dot(a, u_ref[...], preferred_element_type=jnp.float32)
        h = jnp.maximum(h, 0.0)
        norm = jnp.sqrt(jnp.sum(h * h, axis=1, keepdims=True))
        hn = h / jnp.maximum(norm, 1e-12)
        hn_ref[pl.ds(i * ABLK, ABLK), :] = hn
        yin_ref[pl.ds(i * ABLK, ABLK), :] = jnp.dot(
            h, w2_ref[...], preferred_element_type=jnp.float32)
        adjbf_ref[pl.ds(i * ABLK, ABLK), :] = a.astype(jnp.bfloat16)

    @pl.when(jnp.logical_and(i >= NA, i < NA + NB))
    def _phase_b():
        j = i - NA
        hnb = hn_ref[pl.ds(j * SBLK, SBLK), :]
        s = jax.lax.dot_general(
            hnb, hn_ref[...],
            dimension_numbers=(((1,), (1,)), ((), ())),
            preferred_element_type=jnp.float32)
        s = jnp.where(s < THRESH, 0.0, s)
        rs = jnp.sum(s, axis=1, keepdims=True)
        agg = jnp.dot(s, yin_ref[...], preferred_element_type=jnp.float32)
        y = agg / jnp.maximum(rs, 1e-12)
        y_ref[...] = y
        x_ref[pl.ds(j * SBLK, SBLK), :] = jnp.dot(
            y, dw_ref[...], preferred_element_type=jnp.float32
        ).astype(jnp.bfloat16)
        if z_ref is not None:
            z_ref[...] = (y + yprev_ref[...]) * 0.5

    @pl.when(i >= NA + NB)
    def _phase_c():
        k = i - (NA + NB)
        recon_ref[...] = jnp.dot(
            adjbf_ref[pl.ds(k * CBLK, CBLK), :], x_ref[...],
            preferred_element_type=jnp.float32)


def _chain1_kernel(a_ref, u_ref, w2_ref, dw_ref,
                   y_ref, recon_ref,
                   adjbf_ref, hn_ref, yin_ref, x_ref):
    _chain_body(a_ref, u_ref, w2_ref, dw_ref, None,
                y_ref, recon_ref, None,
                adjbf_ref, hn_ref, yin_ref, x_ref)


def _chain2_kernel(a_ref, u_ref, w2_ref, dw_ref, yprev_ref,
                   y_ref, recon_ref, z_ref,
                   adjbf_ref, hn_ref, yin_ref, x_ref):
    _chain_body(a_ref, u_ref, w2_ref, dw_ref, yprev_ref,
                y_ref, recon_ref, z_ref,
                adjbf_ref, hn_ref, yin_ref, x_ref)


def _chain(adj, u, w2, dw, yprev=None):
    d = dw.shape[1]
    grid = (NA + NB + NC,)
    in_specs = [
        pl.BlockSpec((ABLK, N), lambda i: (jnp.minimum(i, NA - 1), 0)),
        pl.BlockSpec((N, HID), lambda i: (0, 0)),
        pl.BlockSpec((HID, O), lambda i: (0, 0)),
        pl.BlockSpec((O, d), lambda i: (0, 0)),
    ]
    out_specs = [
        pl.BlockSpec((SBLK, O),
                     lambda i: (jnp.clip(i - NA, 0, NB - 1), 0)),
        pl.BlockSpec((CBLK, d),
                     lambda i: (jnp.clip(i - NA - NB, 0, NC - 1), 0)),
    ]
    out_shape = [
        jax.ShapeDtypeStruct((N, O), jnp.float32),
        jax.ShapeDtypeStruct((N, d), jnp.float32),
    ]
    scratch_shapes = [
        pltpu.VMEM((N, N), jnp.bfloat16),
        pltpu.VMEM((N, HID), jnp.float32),
        pltpu.VMEM((N, O), jnp.float32),
        pltpu.VMEM((N, d), jnp.bfloat16),
    ]
    args = [adj, u, w2, dw]
    body = _chain1_kernel
    if yprev is not None:
        in_specs.append(
            pl.BlockSpec((SBLK, O),
                         lambda i: (jnp.clip(i - NA, 0, NB - 1), 0)))
        out_specs.append(
            pl.BlockSpec((SBLK, O),
                         lambda i: (jnp.clip(i - NA, 0, NB - 1), 0)))
        out_shape.append(jax.ShapeDtypeStruct((N, O), jnp.float32))
        args.append(yprev)
        body = _chain2_kernel
    return pl.pallas_call(
        body,
        grid=grid,
        in_specs=in_specs,
        out_specs=out_specs,
        out_shape=out_shape,
        scratch_shapes=scratch_shapes,
    )(*args)


def kernel(feat1, feat2, adj_spatial1, adj_spatial2,
           e1w1, e1w2, e2w1, e2w2, d1w, d2w):
    u1, u2 = _u(feat1, feat2, e1w1, e2w1)
    y1, recon1 = _chain(adj_spatial1, u1, e1w2, d1w)
    y2, recon2, z = _chain(adj_spatial2, u2, e2w2, d2w, y1)
    return (y1, y2, z, recon1, recon2)
